# baseline (speedup 1.0000x reference)
import functools

import jax
import jax.numpy as jnp
from jax import lax
from jax.experimental import pallas as pl
from jax.experimental.pallas import tpu as pltpu
from jax.experimental.pallas import tpu_sc as plsc

D = 64
NC = 2
NS = 16
NW = NC * NS
G = 128
NB = 4
NT = 2


def _embed_call(n_tok, n_seq, V):
    n_units = n_tok // G * n_seq
    u_per_w = n_units // NW
    assert u_per_w % NB == 0
    tb_per_s = n_tok // G
    mesh = plsc.VectorSubcoreMesh(
        core_axis_name="c", subcore_axis_name="s",
        num_cores=NC, num_subcores=NS)

    @functools.partial(
        pl.kernel,
        mesh=mesh,
        compiler_params=pltpu.CompilerParams(
            use_tc_tiling_on_sc=False, needs_layout_passes=False),
        out_type=jax.ShapeDtypeStruct((n_units * 8, 8, G), jnp.float32),
        scratch_types=[
            pltpu.VMEM((u_per_w, G), jnp.int32),
            pltpu.VMEM((NB, G, D), jnp.float32),
            pltpu.VMEM((NT, 8, 8, G), jnp.float32),
            [pltpu.SemaphoreType.DMA] * NB,
            [pltpu.SemaphoreType.DMA] * NT,
        ],
    )
    def k(idx_hbm, tab_hbm, out_hbm, idx_v, buf, tout, sems_g, sems_s):
        wid = lax.axis_index("s") * NC + lax.axis_index("c")
        base_u = wid * u_per_w
        pltpu.sync_copy(idx_hbm.at[wid], idx_v)

        rowv = [lax.iota(jnp.int32, 16) + lj * 16 for lj in range(8)]

        def issue_gather(un, slot):
            pltpu.async_copy(
                tab_hbm.at[idx_v.at[un]], buf.at[slot], sems_g[slot])

        def wait_gather(slot):
            pltpu.make_async_copy(
                tab_hbm.at[pl.ds(0, G)], buf.at[slot], sems_g[slot]).wait()

        def transpose(p, q):
            for d in range(D):
                col = jnp.full((16,), d, jnp.int32)
                for lj in range(8):
                    vec = plsc.load_gather(buf.at[p], [rowv[lj], col])
                    tout[q, d // 8, d % 8, pl.ds(lj * 16, 16)] = vec

        def issue_store(un, q):
            ug = base_u + un
            s = ug // tb_per_s
            tb = ug % tb_per_s
            for g in range(8):
                pltpu.async_copy(
                    tout.at[q, g],
                    out_hbm.at[(s * 8 + g) * tb_per_s + tb],
                    sems_s[q])

        def wait_store(q):
            pltpu.make_async_copy(
                tout.at[q], out_hbm.at[pl.ds(0, 8)], sems_s[q]).wait()

        for un in range(NB - 1):
            issue_gather(un, un)

        def body(i, _):
            for p in range(NB):
                un = NB * i + p
                wait_gather(p)

                @pl.when(un + NB - 1 < u_per_w)
                def _():
                    issue_gather(un + NB - 1, (p + NB - 1) % NB)

                q = p % NT

                @pl.when(un >= NT)
                def _():
                    wait_store(q)

                transpose(p, q)
                issue_store(un, q)
            return 0

        lax.fori_loop(0, u_per_w // NB, body, 0)
        for q in range(NT):
            wait_store(q)

    return k


def kernel(x, weight):
    n_tok, n_seq = x.shape
    xu = x.astype(jnp.int32).T.reshape(NW, -1, G)
    outk = _embed_call(n_tok, n_seq, weight.shape[0])(xu, weight)
    out5 = outk.reshape(n_seq, 8, n_tok // G, 8, G)
    fin = out5.transpose(0, 1, 3, 2, 4).reshape(n_seq, D, n_tok)
    return fin.transpose(2, 0, 1)


# speedup vs baseline: 2.0338x; 2.0338x over previous
import functools

import jax
import jax.numpy as jnp
from jax import lax
from jax.experimental import pallas as pl
from jax.experimental.pallas import tpu as pltpu
from jax.experimental.pallas import tpu_sc as plsc

D = 64
NC = 2
NS = 16
NW = NC * NS
G = 128
NB = 4
NT = 2
TP = 129


def _embed_call(n_tok, n_seq, V):
    n_units = n_tok // G * n_seq
    u_per_w = n_units // NW
    assert u_per_w % NB == 0
    tb_per_s = n_tok // G
    mesh = plsc.VectorSubcoreMesh(
        core_axis_name="c", subcore_axis_name="s",
        num_cores=NC, num_subcores=NS)

    @functools.partial(
        pl.kernel,
        mesh=mesh,
        compiler_params=pltpu.CompilerParams(needs_layout_passes=False),
        out_type=jax.ShapeDtypeStruct((n_units * 8, 8, G), jnp.float32),
        scratch_types=[
            pltpu.VMEM((u_per_w, G), jnp.int32),
            pltpu.VMEM((NB, G), jnp.int32),
            pltpu.VMEM((NB, G, 2 * D), jnp.float32),
            pltpu.VMEM((NT * D, G), jnp.float32),
            [pltpu.SemaphoreType.DMA] * NB,
            [pltpu.SemaphoreType.DMA] * NT,
        ],
    )
    def k(idx_hbm, tab_hbm, out_hbm, idx_v, pring, buf, tout,
          sems_g, sems_s):
        wid = lax.axis_index("s") * NC + lax.axis_index("c")
        base_u = wid * u_per_w
        pltpu.sync_copy(idx_hbm.at[wid], idx_v)

        iota16 = lax.iota(jnp.int32, 16)
        rowv = [iota16 + 16 * lj for lj in range(8)]
        dconst = [(iota16 + o) & 15 for o in range(16)]

        def fill_pidx_and_gather(un, slot):
            for lj in range(8):
                seg = idx_v[un, pl.ds(lj * 16, 16)]
                pring[slot, pl.ds(lj * 16, 16)] = seg >> 1
            pltpu.async_copy(
                tab_hbm.at[pring.at[slot]], buf.at[slot], sems_g[slot])

        def wait_gather(slot):
            pltpu.make_async_copy(
                tab_hbm.at[pl.ds(0, G)], buf.at[slot], sems_g[slot]).wait()

        def transpose(un, p, q):
            hv = []
            for lj in range(8):
                seg = idx_v[un, pl.ds(lj * 16, 16)]
                hv.append((seg & 1) << 6)
            def kd_body(kd, _):
                for o in range(16):
                    dk = dconst[o] + 16 * kd
                    drow = dk + (q * D)
                    for lj in range(8):
                        vec = plsc.load_gather(
                            buf.at[p], [rowv[lj], hv[lj] + dk])
                        plsc.store_scatter(tout, [drow, rowv[lj]], vec)
                return 0

            lax.fori_loop(0, 4, kd_body, 0)

        def issue_store(un, q):
            ug = base_u + un
            s = ug // tb_per_s
            tb = ug % tb_per_s
            for g in range(8):
                pltpu.async_copy(
                    tout.at[pl.ds(q * D + 8 * g, 8), pl.ds(0, G)],
                    out_hbm.at[(s * 8 + g) * tb_per_s + tb],
                    sems_s[q])

        def wait_store(q):
            for g in range(8):
                pltpu.make_async_copy(
                    tout.at[pl.ds(q * D + 8 * g, 8), pl.ds(0, G)],
                    out_hbm.at[0], sems_s[q]).wait()

        for un in range(NB - 1):
            fill_pidx_and_gather(un, un)

        def body(i, _):
            for p in range(NB):
                un = NB * i + p
                wait_gather(p)

                @pl.when(un + NB - 1 < u_per_w)
                def _():
                    fill_pidx_and_gather(un + NB - 1, (p + NB - 1) % NB)

                q = p % NT

                @pl.when(un >= NT)
                def _():
                    wait_store(q)

                transpose(un, p, q)
                issue_store(un, q)
            return 0

        lax.fori_loop(0, u_per_w // NB, body, 0)
        for q in range(NT):
            wait_store(q)

    return k


def kernel(x, weight):
    n_tok, n_seq = x.shape
    xu = x.astype(jnp.int32).T.reshape(NW, -1, G)
    wp = weight.reshape(-1, 2 * D)
    outk = _embed_call(n_tok, n_seq, weight.shape[0])(xu, wp)
    out5 = outk.reshape(n_seq, 8, n_tok // G, 8, G)
    fin = out5.transpose(0, 1, 3, 2, 4).reshape(n_seq, D, n_tok)
    return fin.transpose(2, 0, 1)


# speedup vs baseline: 2.0936x; 1.0294x over previous
import functools

import jax
import jax.numpy as jnp
from jax import lax
from jax.experimental import pallas as pl
from jax.experimental.pallas import tpu as pltpu
from jax.experimental.pallas import tpu_sc as plsc

D = 64
NC = 2
NS = 16
NW = NC * NS
G = 128
NB = 4
NT = 2
TP = 129


def _embed_call(n_tok, n_seq, V):
    n_units = n_tok // G * n_seq
    u_per_w = n_units // NW
    assert u_per_w % NB == 0
    tb_per_s = n_tok // G
    mesh = plsc.VectorSubcoreMesh(
        core_axis_name="c", subcore_axis_name="s",
        num_cores=NC, num_subcores=NS)

    @functools.partial(
        pl.kernel,
        mesh=mesh,
        compiler_params=pltpu.CompilerParams(needs_layout_passes=False),
        out_type=jax.ShapeDtypeStruct((n_units * 8, 8, G), jnp.float32),
        scratch_types=[
            pltpu.VMEM((u_per_w, G), jnp.int32),
            pltpu.VMEM((NB, G), jnp.int32),
            pltpu.VMEM((NB, G, 2 * D), jnp.float32),
            pltpu.VMEM((NT * D, G), jnp.float32),
            [pltpu.SemaphoreType.DMA] * NB,
            [pltpu.SemaphoreType.DMA] * NT,
        ],
    )
    def k(idx_hbm, tab_hbm, out_hbm, idx_v, pring, buf, tout,
          sems_g, sems_s):
        wid = lax.axis_index("s") * NC + lax.axis_index("c")
        base_u = wid * u_per_w
        pltpu.sync_copy(idx_hbm.at[wid], idx_v)

        iota16 = lax.iota(jnp.int32, 16)
        rowv = [iota16 + 16 * lj for lj in range(8)]
        dconst = [(iota16 + o) & 15 for o in range(16)]

        def fill_pidx_and_gather(un, slot):
            for lj in range(8):
                seg = idx_v[un, pl.ds(lj * 16, 16)]
                pring[slot, pl.ds(lj * 16, 16)] = seg >> 1
            pltpu.async_copy(
                tab_hbm.at[pring.at[slot]], buf.at[slot], sems_g[slot])

        def wait_gather(slot):
            pltpu.make_async_copy(
                tab_hbm.at[pl.ds(0, G)], buf.at[slot], sems_g[slot]).wait()

        def transpose(un, p, q):
            hv = []
            for lj in range(8):
                seg = idx_v[un, pl.ds(lj * 16, 16)]
                hv.append((seg & 1) << 6)
            def kd_body(kd, _):
                for o in range(16):
                    dk = dconst[o] + 16 * kd
                    drow = dk + (q * D)
                    for lj in range(8):
                        vec = plsc.load_gather(
                            buf.at[p], [rowv[lj], hv[lj] + dk])
                        plsc.store_scatter(tout, [drow, rowv[lj]], vec)
                return 0

            lax.fori_loop(0, 4, kd_body, 0)

        def issue_store(un, q):
            ug = base_u + un
            s = ug // tb_per_s
            tb = ug % tb_per_s
            for g in range(8):
                pltpu.async_copy(
                    tout.at[pl.ds(q * D + 8 * g, 8), pl.ds(0, G)],
                    out_hbm.at[(s * 8 + g) * tb_per_s + tb],
                    sems_s[q])

        def wait_store(q):
            for g in range(8):
                pltpu.make_async_copy(
                    tout.at[pl.ds(q * D + 8 * g, 8), pl.ds(0, G)],
                    out_hbm.at[0], sems_s[q]).wait()

        for un in range(NB - 1):
            fill_pidx_and_gather(un, un)

        def body(i, _):
            for p in range(NB):
                un = NB * i + p
                wait_gather(p)

                @pl.when(un + NB - 1 < u_per_w)
                def _():
                    fill_pidx_and_gather(un + NB - 1, (p + NB - 1) % NB)

                q = p % NT

                @pl.when(un >= NT)
                def _():
                    wait_store(q)

                transpose(un, p, q)
                issue_store(un, q)
            return 0

        lax.fori_loop(0, u_per_w // NB, body, 0)
        for q in range(NT):
            wait_store(q)

    return k


def _format_call(V):
    n_cols = V // G
    n_extra = V % G
    base_cnt = n_cols // NW
    n_more = n_cols % NW
    max_cnt = base_cnt + (1 if n_more else 0)
    n_iter = (max_cnt + 1) // 2
    mesh = plsc.VectorSubcoreMesh(
        core_axis_name="c", subcore_axis_name="s",
        num_cores=NC, num_subcores=NS)

    @functools.partial(
        pl.kernel,
        mesh=mesh,
        compiler_params=pltpu.CompilerParams(needs_layout_passes=False),
        out_type=jax.ShapeDtypeStruct((V // 2, 2 * D), jnp.float32),
        scratch_types=[
            pltpu.VMEM((2, 8, 8, G), jnp.float32),
            pltpu.VMEM((2 * D, G), jnp.float32),
            pltpu.VMEM((n_extra // 2, 2 * D), jnp.float32),
            [pltpu.SemaphoreType.DMA] * 2,
            [pltpu.SemaphoreType.DMA] * 2,
        ],
    )
    def k(wq_hbm, wtail_hbm, wp_hbm, qbuf, pbuf, tailbuf, sems_i, sems_o):
        wid = lax.axis_index("s") * NC + lax.axis_index("c")
        start = wid * base_cnt + jnp.minimum(wid, n_more)
        cnt = base_cnt + (wid < n_more).astype(jnp.int32)

        iota16 = lax.iota(jnp.int32, 16)
        iota2 = iota16 * 2
        dconst = [(iota16 + o) & 15 for o in range(16)]
        rowvP = [iota16 + 16 * plb for plb in range(8)]

        def issue_inputs(un, p):
            tc = start + un
            off = pl.multiple_of(tc * G, G)
            for g in range(8):
                pltpu.async_copy(
                    wq_hbm.at[pl.ds(8 * g, 8), pl.ds(off, G)],
                    qbuf.at[p, g], sems_i[p])

        def wait_inputs(p):
            for g in range(8):
                pltpu.make_async_copy(
                    wq_hbm.at[pl.ds(0, 8), pl.ds(0, G)],
                    qbuf.at[p, g], sems_i[p]).wait()

        def transpose(p):
            def cb_body(cb, _):
                h = cb // 4
                d0 = (cb % 4) * 16
                for o in range(16):
                    dvec = dconst[o] + d0
                    gv = dvec >> 3
                    rv = dvec & 7
                    cov = dconst[o] + cb * 16
                    for plb in range(4):
                        colv = iota2 + (2 * (plb * 16) + h)
                        vec = plsc.load_gather(qbuf.at[p], [gv, rv, colv])
                        plsc.store_scatter(
                            pbuf, [rowvP[plb + 4 * p], cov], vec)
                return 0

            lax.fori_loop(0, 8, cb_body, 0)

        def issue_store(un, p):
            tc = start + un
            off = pl.multiple_of(tc * D, 8)
            pltpu.async_copy(
                pbuf.at[pl.ds(p * D, D), pl.ds(0, G)],
                wp_hbm.at[pl.ds(off, D)], sems_o[p])

        def wait_store(p):
            pltpu.make_async_copy(
                pbuf.at[pl.ds(p * D, D), pl.ds(0, G)],
                wp_hbm.at[pl.ds(0, D)], sems_o[p]).wait()

        for un in range(2):
            @pl.when(un < cnt)
            def _():
                issue_inputs(un, un)

        def body(i, _):
            for p in range(2):
                un = 2 * i + p

                @pl.when(un < cnt)
                def _():
                    @pl.when(un >= 2)
                    def _():
                        wait_store(p)

                    wait_inputs(p)
                    transpose(p)
                    issue_store(un, p)

                    @pl.when(un + 2 < cnt)
                    def _():
                        issue_inputs(un + 2, p)
            return 0

        lax.fori_loop(0, n_iter, body, 0)
        for p in range(2):
            wait_store(p)

        if n_extra:
            @pl.when(wid == NW - 1)
            def _():
                pltpu.sync_copy(wtail_hbm, tailbuf)
                pltpu.sync_copy(
                    tailbuf, wp_hbm.at[pl.ds(n_cols * D, n_extra // 2)])

    return k


def kernel(x, weight):
    n_tok, n_seq = x.shape
    xu = x.astype(jnp.int32).T.reshape(NW, -1, G)
    n_cols = weight.shape[0] // G
    wtail = weight[n_cols * G:].reshape(-1, 2 * D)
    wp = _format_call(weight.shape[0])(weight.T, wtail)
    outk = _embed_call(n_tok, n_seq, weight.shape[0])(xu, wp)
    out5 = outk.reshape(n_seq, 8, n_tok // G, 8, G)
    fin = out5.transpose(0, 1, 3, 2, 4).reshape(n_seq, D, n_tok)
    return fin.transpose(2, 0, 1)
